# initial kernel scaffold (unmeasured)
import jax
import jax.numpy as jnp
from jax import lax
from jax.experimental import pallas as pl
from jax.experimental.pallas import tpu as pltpu


def kernel(
    x,
):
    def body(*refs):
        pass

    out_shape = jax.ShapeDtypeStruct(..., jnp.float32)
    return pl.pallas_call(body, out_shape=out_shape)(...)


# baseline (device time: 2129678 ns/iter reference)
import jax
import jax.numpy as jnp
from jax import lax
from jax.experimental import pallas as pl
from jax.experimental.pallas import tpu as pltpu


def kernel(x):
    m, n = x.shape

    def body(x_ref, out_ref, send_sem, recv_sem, local_sem):
        my_x = lax.axis_index("x")
        my_y = lax.axis_index("y")
        my_z = lax.axis_index("z")
        nbr = (my_x, 1 - my_y, my_z)

        barrier_sem = pltpu.get_barrier_semaphore()
        pl.semaphore_signal(
            barrier_sem, inc=1, device_id=nbr,
            device_id_type=pl.DeviceIdType.MESH,
        )
        pl.semaphore_wait(barrier_sem, 1)

        local = pltpu.make_async_copy(
            x_ref, out_ref.at[pl.ds(my_y * m, m)], local_sem
        )
        local.start()

        rdma = pltpu.make_async_remote_copy(
            src_ref=x_ref,
            dst_ref=out_ref.at[pl.ds(my_y * m, m)],
            send_sem=send_sem,
            recv_sem=recv_sem,
            device_id=nbr,
            device_id_type=pl.DeviceIdType.MESH,
        )
        rdma.start()

        local.wait()
        rdma.wait()

    return pl.pallas_call(
        body,
        out_shape=jax.ShapeDtypeStruct((2 * m, n), x.dtype),
        in_specs=[pl.BlockSpec(memory_space=pl.ANY)],
        out_specs=pl.BlockSpec(memory_space=pl.ANY),
        scratch_shapes=[
            pltpu.SemaphoreType.DMA,
            pltpu.SemaphoreType.DMA,
            pltpu.SemaphoreType.DMA,
        ],
        compiler_params=pltpu.CompilerParams(collective_id=0),
    )(x)


# device time: 2128278 ns/iter; 1.0007x vs baseline; 1.0007x over previous
import jax
import jax.numpy as jnp
from jax import lax
from jax.experimental import pallas as pl
from jax.experimental.pallas import tpu as pltpu


N_CHUNKS = 16


def kernel(x):
    m, n = x.shape
    rows = m // N_CHUNKS

    def body(x_ref, out_ref, send_sems, recv_sems, local_sems):
        my_x = lax.axis_index("x")
        my_y = lax.axis_index("y")
        my_z = lax.axis_index("z")
        nbr = (my_x, 1 - my_y, my_z)

        barrier_sem = pltpu.get_barrier_semaphore()
        pl.semaphore_signal(
            barrier_sem, inc=1, device_id=nbr,
            device_id_type=pl.DeviceIdType.MESH,
        )
        pl.semaphore_wait(barrier_sem, 1)

        rdmas = []
        locals_ = []
        for k in range(N_CHUNKS):
            src = x_ref.at[pl.ds(k * rows, rows)]
            rdma = pltpu.make_async_remote_copy(
                src_ref=src,
                dst_ref=out_ref.at[pl.ds(my_y * m + k * rows, rows)],
                send_sem=send_sems.at[k],
                recv_sem=recv_sems.at[k],
                device_id=nbr,
                device_id_type=pl.DeviceIdType.MESH,
            )
            rdma.start()
            rdmas.append(rdma)
            local = pltpu.make_async_copy(
                src, out_ref.at[pl.ds(my_y * m + k * rows, rows)],
                local_sems.at[k],
            )
            local.start()
            locals_.append(local)

        for k in range(N_CHUNKS):
            locals_[k].wait()
            rdmas[k].wait()

    return pl.pallas_call(
        body,
        out_shape=jax.ShapeDtypeStruct((2 * m, n), x.dtype),
        in_specs=[pl.BlockSpec(memory_space=pl.ANY)],
        out_specs=pl.BlockSpec(memory_space=pl.ANY),
        scratch_shapes=[
            pltpu.SemaphoreType.DMA((N_CHUNKS,)),
            pltpu.SemaphoreType.DMA((N_CHUNKS,)),
            pltpu.SemaphoreType.DMA((N_CHUNKS,)),
        ],
        compiler_params=pltpu.CompilerParams(collective_id=0),
    )(x)


# device time: 811894 ns/iter; 2.6231x vs baseline; 2.6214x over previous
import jax
import jax.numpy as jnp
from jax import lax
from jax.experimental import pallas as pl
from jax.experimental.pallas import tpu as pltpu

N_CHUNKS = 16


def kernel(x):
    m, n = x.shape
    rows = m // N_CHUNKS

    def body(x_ref, out_ref, vbuf, send_sems, recv_sems, in_sems, out_sems):
        my_x = lax.axis_index("x")
        my_y = lax.axis_index("y")
        my_z = lax.axis_index("z")
        nbr = (my_x, 1 - my_y, my_z)

        barrier_sem = pltpu.get_barrier_semaphore()
        pl.semaphore_signal(
            barrier_sem, inc=1, device_id=nbr,
            device_id_type=pl.DeviceIdType.MESH,
        )
        pl.semaphore_wait(barrier_sem, 1)

        rdmas = []
        for k in range(N_CHUNKS):
            rdma = pltpu.make_async_remote_copy(
                src_ref=x_ref.at[pl.ds(k * rows, rows)],
                dst_ref=out_ref.at[pl.ds(my_y * m + k * rows, rows)],
                send_sem=send_sems.at[k],
                recv_sem=recv_sems.at[k],
                device_id=nbr,
                device_id_type=pl.DeviceIdType.MESH,
            )
            rdma.start()
            rdmas.append(rdma)

        stores = []
        for k in range(N_CHUNKS):
            slot = k % 2
            if k >= 2:
                stores[k - 2].wait()
            ld = pltpu.make_async_copy(
                x_ref.at[pl.ds(k * rows, rows)], vbuf.at[slot],
                in_sems.at[slot],
            )
            ld.start()
            ld.wait()
            st = pltpu.make_async_copy(
                vbuf.at[slot],
                out_ref.at[pl.ds(my_y * m + k * rows, rows)],
                out_sems.at[slot],
            )
            st.start()
            stores.append(st)
        stores[-2].wait()
        stores[-1].wait()

        for rdma in rdmas:
            rdma.wait()

    return pl.pallas_call(
        body,
        out_shape=jax.ShapeDtypeStruct((2 * m, n), x.dtype),
        in_specs=[pl.BlockSpec(memory_space=pl.ANY)],
        out_specs=pl.BlockSpec(memory_space=pl.ANY),
        scratch_shapes=[
            pltpu.VMEM((2, rows, n), x.dtype),
            pltpu.SemaphoreType.DMA((N_CHUNKS,)),
            pltpu.SemaphoreType.DMA((N_CHUNKS,)),
            pltpu.SemaphoreType.DMA((2,)),
            pltpu.SemaphoreType.DMA((2,)),
        ],
        compiler_params=pltpu.CompilerParams(collective_id=0),
    )(x)


# device time: 545592 ns/iter; 3.9034x vs baseline; 1.4881x over previous
import jax
import jax.numpy as jnp
from jax import lax
from jax.experimental import pallas as pl
from jax.experimental.pallas import tpu as pltpu

N_LOCAL_CHUNKS = 16


def kernel(x):
    m, n = x.shape
    qrows = m // 4
    hq = qrows // 2
    lrows = m // N_LOCAL_CHUNKS

    def body(x_ref, out_ref, vbuf, send_sems, recv_sems, in_sems, out_sems):
        my_x = lax.axis_index("x")
        my_y = lax.axis_index("y")
        my_z = lax.axis_index("z")
        y_nbr = (my_x, 1 - my_y, my_z)
        x_nbr = (1 - my_x, my_y, my_z)
        z_nbr = (my_x, my_y, 1 - my_z)

        q_me = 2 * my_x + my_z
        q_x = 2 * (1 - my_x) + my_z
        q_z = 2 * my_x + (1 - my_z)
        q_d = 2 * (1 - my_x) + (1 - my_z)

        f_base = (1 - my_y) * m

        barrier_sem = pltpu.get_barrier_semaphore()
        for nbr in (y_nbr, x_nbr, z_nbr):
            pl.semaphore_signal(
                barrier_sem, inc=1, device_id=nbr,
                device_id_type=pl.DeviceIdType.MESH,
            )
        pl.semaphore_wait(barrier_sem, 3)

        def remote(src, dst, k, dev):
            return pltpu.make_async_remote_copy(
                src_ref=src, dst_ref=dst,
                send_sem=send_sems.at[k], recv_sem=recv_sems.at[k],
                device_id=dev, device_id_type=pl.DeviceIdType.MESH,
            )

        r1 = remote(
            x_ref.at[pl.ds(q_me * qrows, qrows)],
            out_ref.at[pl.ds(f_base + q_me * qrows, qrows)],
            0, y_nbr,
        )
        r1.start()

        stores = []
        for k in range(N_LOCAL_CHUNKS):
            slot = k % 2
            if k >= 2:
                stores[k - 2].wait()
            ld = pltpu.make_async_copy(
                x_ref.at[pl.ds(k * lrows, lrows)], vbuf.at[slot],
                in_sems.at[slot],
            )
            ld.start()
            ld.wait()
            st = pltpu.make_async_copy(
                vbuf.at[slot],
                out_ref.at[pl.ds(my_y * m + k * lrows, lrows)],
                out_sems.at[slot],
            )
            st.start()
            stores.append(st)

        r1.wait_recv()

        fq_me = out_ref.at[pl.ds(f_base + q_me * qrows, qrows)]
        r2 = remote(fq_me, fq_me, 1, x_nbr)
        r2.start()
        r3 = remote(fq_me, fq_me, 2, z_nbr)
        r3.start()

        r2.wait_recv()
        r4b = remote(
            out_ref.at[pl.ds(f_base + q_x * qrows + hq, hq)],
            out_ref.at[pl.ds(f_base + q_x * qrows + hq, hq)],
            3, z_nbr,
        )
        r4b.start()
        r3.wait_recv()
        r4a = remote(
            out_ref.at[pl.ds(f_base + q_z * qrows, hq)],
            out_ref.at[pl.ds(f_base + q_z * qrows, hq)],
            4, x_nbr,
        )
        r4a.start()

        stores[-2].wait()
        stores[-1].wait()
        for r in (r1, r2, r3, r4a, r4b):
            r.wait_send()
        r4a.wait_recv()
        r4b.wait_recv()

    return pl.pallas_call(
        body,
        out_shape=jax.ShapeDtypeStruct((2 * m, n), x.dtype),
        in_specs=[pl.BlockSpec(memory_space=pl.ANY)],
        out_specs=pl.BlockSpec(memory_space=pl.ANY),
        scratch_shapes=[
            pltpu.VMEM((2, lrows, n), x.dtype),
            pltpu.SemaphoreType.DMA((5,)),
            pltpu.SemaphoreType.DMA((5,)),
            pltpu.SemaphoreType.DMA((2,)),
            pltpu.SemaphoreType.DMA((2,)),
        ],
        compiler_params=pltpu.CompilerParams(collective_id=0),
    )(x)
